# Initial kernel scaffold; baseline (speedup 1.0000x reference)
import jax
import jax.numpy as jnp
from jax.experimental import pallas as pl


def kernel(x, indices, pe):
    raise NotImplementedError("write your pallas kernel here")


# speedup vs baseline: 1.2294x; 1.2294x over previous
import functools

import jax
import jax.numpy as jnp
from jax import lax
from jax.experimental import pallas as pl
from jax.experimental.pallas import tpu as pltpu
from jax.experimental.pallas import tpu_sc as plsc

B, P, D = 4, 8192, 768
N_ROWS = B * P
NC, NS, L = 2, 16, 16
NW = NC * NS
ROWS_PER_W = N_ROWS // NW
C = 32
NCHUNK = ROWS_PER_W // C
VPR = D // L


def _sc_body(x_hbm, idx_hbm, pe_hbm, out_hbm, idx_v, x_v, pe_v, sem):
    wid = lax.axis_index("s") * NC + lax.axis_index("c")
    base0 = wid * ROWS_PER_W

    def chunk(i, carry):
        base = base0 + i * C
        pltpu.sync_copy(idx_hbm.at[pl.ds(base, C)], idx_v)
        gather = pltpu.async_copy(pe_hbm.at[idx_v], pe_v, sem)
        pltpu.sync_copy(x_hbm.at[pl.ds(base, C)], x_v)
        gather.wait()

        def row(r, rcarry):
            for c in range(VPR):
                sl = pl.ds(c * L, L)
                x_v[r, sl] = x_v[r, sl] + pe_v[r, sl]
            return rcarry

        lax.fori_loop(0, C, row, 0)
        pltpu.sync_copy(x_v, out_hbm.at[pl.ds(base, C)])
        return carry

    lax.fori_loop(0, NCHUNK, chunk, 0)


@jax.jit
def _pe_add(x2d, idx1d, pe2d):
    mesh = plsc.VectorSubcoreMesh(
        core_axis_name="c", subcore_axis_name="s", num_cores=NC, num_subcores=NS
    )
    f = pl.kernel(
        _sc_body,
        out_type=jax.ShapeDtypeStruct((N_ROWS, D), jnp.float32),
        mesh=mesh,
        scratch_types=[
            pltpu.VMEM((C,), jnp.int32),
            pltpu.VMEM((C, D), jnp.float32),
            pltpu.VMEM((C, D), jnp.float32),
            pltpu.SemaphoreType.DMA,
        ],
    )
    return f(x2d, idx1d, pe2d)


def kernel(x, indices, pe):
    out = _pe_add(
        x.reshape(N_ROWS, D), indices.reshape(N_ROWS), pe.reshape(P, D)
    )
    return out.reshape(B, P, D)


# speedup vs baseline: 1.6223x; 1.3195x over previous
import jax
import jax.numpy as jnp
from jax import lax
from jax.experimental import pallas as pl
from jax.experimental.pallas import tpu as pltpu
from jax.experimental.pallas import tpu_sc as plsc

B, P, D = 4, 8192, 768
N_ROWS = B * P
NC, NS, L = 2, 16, 16
NW = NC * NS
ROWS_PER_W = N_ROWS // NW
C = 32
NCHUNK = ROWS_PER_W // C
NK2 = NCHUNK // 2
VPR = D // L


def _sc_body(x_hbm, idx_hbm, pe_hbm, out_hbm, idx_c0, idx_c1, x0, x1, pe0, pe1,
             sem_in0, sem_in1, sem_out0, sem_out1):
    wid = lax.axis_index("s") * NC + lax.axis_index("c")
    base0 = wid * ROWS_PER_W

    idx_cs = (idx_c0, idx_c1)
    xs = (x0, x1)
    pes = (pe0, pe1)
    sems_in = (sem_in0, sem_in1)
    sems_out = (sem_out0, sem_out1)

    def issue(i, b):
        base = base0 + i * C
        pltpu.sync_copy(idx_hbm.at[pl.ds(base, C)], idx_cs[b])
        pltpu.async_copy(pe_hbm.at[idx_cs[b]], pes[b], sems_in[b])
        pltpu.async_copy(x_hbm.at[pl.ds(base, C)], xs[b], sems_in[b])

    def drain_in(b):
        pltpu.make_async_copy(x_hbm.at[pl.ds(0, C)], xs[b], sems_in[b]).wait()
        pltpu.make_async_copy(x_hbm.at[pl.ds(0, C)], pes[b], sems_in[b]).wait()

    def drain_out(b):
        pltpu.make_async_copy(xs[b], out_hbm.at[pl.ds(0, C)], sems_out[b]).wait()

    def add_chunk(b):
        x_v, pe_v = xs[b], pes[b]

        @plsc.parallel_loop(0, C, step=1, unroll=2)
        def _row(r):
            for c in range(VPR):
                sl = pl.ds(c * L, L)
                x_v[r, sl] = x_v[r, sl] + pe_v[r, sl]

    def store(i, b):
        pltpu.async_copy(xs[b], out_hbm.at[pl.ds(base0 + i * C, C)], sems_out[b])

    issue(0, 0)

    def body(k, carry):
        @pl.when(k > 0)
        def _():
            drain_out(1)

        issue(2 * k + 1, 1)
        drain_in(0)
        add_chunk(0)
        store(2 * k, 0)
        drain_in(1)
        add_chunk(1)
        store(2 * k + 1, 1)

        @pl.when(k < NK2 - 1)
        def _():
            drain_out(0)
            issue(2 * k + 2, 0)

        return carry

    lax.fori_loop(0, NK2, body, 0)
    drain_out(0)
    drain_out(1)


@jax.jit
def _pe_add(x2d, idx1d, pe2d):
    mesh = plsc.VectorSubcoreMesh(
        core_axis_name="c", subcore_axis_name="s", num_cores=NC, num_subcores=NS
    )
    f = pl.kernel(
        _sc_body,
        out_type=jax.ShapeDtypeStruct((N_ROWS, D), jnp.float32),
        mesh=mesh,
        scratch_types=[
            pltpu.VMEM((C,), jnp.int32),
            pltpu.VMEM((C,), jnp.int32),
            pltpu.VMEM((C, D), jnp.float32),
            pltpu.VMEM((C, D), jnp.float32),
            pltpu.VMEM((C, D), jnp.float32),
            pltpu.VMEM((C, D), jnp.float32),
            pltpu.SemaphoreType.DMA,
            pltpu.SemaphoreType.DMA,
            pltpu.SemaphoreType.DMA,
            pltpu.SemaphoreType.DMA,
        ],
    )
    return f(x2d, idx1d, pe2d)


def kernel(x, indices, pe):
    out = _pe_add(
        x.reshape(N_ROWS, D), indices.reshape(N_ROWS), pe.reshape(P, D)
    )
    return out.reshape(B, P, D)


# speedup vs baseline: 1.9934x; 1.2287x over previous
import jax
import jax.numpy as jnp
from jax import lax
from jax.experimental import pallas as pl
from jax.experimental.pallas import tpu as pltpu
from jax.experimental.pallas import tpu_sc as plsc

B, P, D = 4, 8192, 768
N_ROWS = B * P
NC, NS, L = 2, 16, 16
NW = NC * NS
ROWS_PER_W = N_ROWS // NW
C = 16
NCHUNK = ROWS_PER_W // C
NSLOT = 4
VPR = D // L


def _sc_body(x_hbm, idx_hbm, pe_hbm, out_hbm, idx_v, xs, pes,
             sems_in, sems_out):
    wid = lax.axis_index("s") * NC + lax.axis_index("c")
    base0 = wid * ROWS_PER_W
    pltpu.sync_copy(idx_hbm.at[pl.ds(base0, ROWS_PER_W)], idx_v)

    def issue(j, b):
        pltpu.async_copy(
            pe_hbm.at[idx_v.at[pl.ds(j * C, C)]], pes[b], sems_in[b]
        )
        pltpu.async_copy(
            x_hbm.at[pl.ds(base0 + j * C, C)], xs[b], sems_in[b]
        )

    def drain_in(b):
        pltpu.make_async_copy(x_hbm.at[pl.ds(0, C)], xs[b], sems_in[b]).wait()
        pltpu.make_async_copy(x_hbm.at[pl.ds(0, C)], pes[b], sems_in[b]).wait()

    def drain_out(b):
        pltpu.make_async_copy(xs[b], out_hbm.at[pl.ds(0, C)], sems_out[b]).wait()

    def add_chunk(b):
        x_v, pe_v = xs[b], pes[b]

        @plsc.parallel_loop(0, C, step=1, unroll=2)
        def _row(r):
            for c in range(VPR):
                sl = pl.ds(c * L, L)
                x_v[r, sl] = x_v[r, sl] + pe_v[r, sl]

    def store(j, b):
        pltpu.async_copy(xs[b], out_hbm.at[pl.ds(base0 + j * C, C)], sems_out[b])

    for s in range(NSLOT - 1):
        issue(s, s)

    def body(k, carry):
        for s in range(NSLOT):
            j = NSLOT * k + s
            t = (s + NSLOT - 1) % NSLOT

            @pl.when((j >= 1) & (j < NCHUNK - NSLOT + 1))
            def _():
                drain_out(t)

            @pl.when(j < NCHUNK - NSLOT + 1)
            def _():
                issue(j + NSLOT - 1, t)

            drain_in(s)
            add_chunk(s)
            store(j, s)
        return carry

    lax.fori_loop(0, NCHUNK // NSLOT, body, 0)
    for s in range(NSLOT):
        drain_out(s)


@jax.jit
def _pe_add(x2d, idx1d, pe2d):
    mesh = plsc.VectorSubcoreMesh(
        core_axis_name="c", subcore_axis_name="s", num_cores=NC, num_subcores=NS
    )

    def entry(x_hbm, idx_hbm, pe_hbm, out_hbm, idx_v,
              x0, x1, x2, x3, pe0, pe1, pe2, pe3,
              si0, si1, si2, si3, so0, so1, so2, so3):
        _sc_body(x_hbm, idx_hbm, pe_hbm, out_hbm, idx_v,
                 (x0, x1, x2, x3), (pe0, pe1, pe2, pe3),
                 (si0, si1, si2, si3), (so0, so1, so2, so3))

    f = pl.kernel(
        entry,
        out_type=jax.ShapeDtypeStruct((N_ROWS, D), jnp.float32),
        mesh=mesh,
        scratch_types=[pltpu.VMEM((ROWS_PER_W,), jnp.int32)]
        + [pltpu.VMEM((C, D), jnp.float32)] * (2 * NSLOT)
        + [pltpu.SemaphoreType.DMA] * (2 * NSLOT),
    )
    return f(x2d, idx1d, pe2d)


def kernel(x, indices, pe):
    out = _pe_add(
        x.reshape(N_ROWS, D), indices.reshape(N_ROWS), pe.reshape(P, D)
    )
    return out.reshape(B, P, D)
